# Initial kernel scaffold; baseline (speedup 1.0000x reference)
import jax
import jax.numpy as jnp
from jax.experimental import pallas as pl


def kernel(inputs, tables):
    raise NotImplementedError("write your pallas kernel here")


# speedup vs baseline: 1.2143x; 1.2143x over previous
import jax
import jax.numpy as jnp
from jax import lax
from jax.experimental import pallas as pl
from jax.experimental.pallas import tpu as pltpu
from jax.experimental.pallas import tpu_sc as plsc

NUM_FIELDS = 26
VOCAB = 100000
EMB_DIM = 32
BATCH = 16384

NC = 2
NS = 16
L = 16
NW = NC * NS

R = BATCH * NUM_FIELDS
CHUNK = 128
CPW = R // (NW * CHUNK)
NB = 4
NG = CPW // NB


def _body(inp_hbm, table_hbm, out_hbm, idx_v, bufs, gsem0, gsem1, wsem0, wsem1):
    wid = lax.axis_index("s") * NC + lax.axis_index("c")
    row0 = wid * CPW

    pltpu.sync_copy(inp_hbm.at[pl.ds(row0, CPW)], idx_v)

    @pl.loop(0, CPW)
    def _add_offsets(r):
        for c in range(CHUNK // L):
            base = r * CHUNK + c * L
            field = lax.rem(lax.iota(jnp.int32, L) + base, NUM_FIELDS)
            sl = (r, pl.ds(c * L, L))
            idx_v[sl] = idx_v[sl] + field * VOCAB

    gsems = (gsem0, gsem1)
    wsems = (wsem0, wsem1)

    def fire_gathers(g, s):
        for b in range(NB):
            pltpu.async_copy(
                table_hbm.at[idx_v.at[g * NB + b]],
                bufs.at[pl.ds((s * NB + b) * CHUNK, CHUNK)],
                gsems[s])

    def drain_gathers(s):
        pltpu.make_async_copy(
            out_hbm.at[pl.ds(0, NB * CHUNK)],
            bufs.at[pl.ds(s * NB * CHUNK, NB * CHUNK)],
            gsems[s]).wait()

    def fire_wbs(g, s):
        for b in range(NB):
            dst_row = (row0 + g * NB + b) * CHUNK
            pltpu.async_copy(
                bufs.at[pl.ds((s * NB + b) * CHUNK, CHUNK)],
                out_hbm.at[pl.ds(dst_row, CHUNK)],
                wsems[s])

    def drain_wbs(s):
        pltpu.make_async_copy(
            bufs.at[pl.ds(s * NB * CHUNK, NB * CHUNK)],
            out_hbm.at[pl.ds(0, NB * CHUNK)],
            wsems[s]).wait()

    fire_gathers(0, 0)
    fire_gathers(1, 1)

    @pl.loop(0, NG // 2 - 1)
    def _main(k):
        for s in range(2):
            g = 2 * k + s
            drain_gathers(s)
            fire_wbs(g, s)
            drain_wbs(s)
            fire_gathers(g + 2, s)

    for s in range(2):
        g = NG - 2 + s
        drain_gathers(s)
        fire_wbs(g, s)
        drain_wbs(s)


def _sc_gather():
    mesh = plsc.VectorSubcoreMesh(
        core_axis_name="c", subcore_axis_name="s",
        num_cores=NC, num_subcores=NS)
    return pl.kernel(
        _body,
        out_type=jax.ShapeDtypeStruct((R, EMB_DIM), jnp.float32),
        mesh=mesh,
        scratch_types=[
            pltpu.VMEM((CPW, CHUNK), jnp.int32),
            pltpu.VMEM((2 * NB * CHUNK, EMB_DIM), jnp.float32),
            pltpu.SemaphoreType.DMA,
            pltpu.SemaphoreType.DMA,
            pltpu.SemaphoreType.DMA,
            pltpu.SemaphoreType.DMA,
        ],
        compiler_params=pltpu.CompilerParams(use_tc_tiling_on_sc=False),
    )


def kernel(inputs, tables):
    inp2 = inputs.astype(jnp.int32).reshape(R // CHUNK, CHUNK)
    tables_flat = tables.reshape(NUM_FIELDS * VOCAB, EMB_DIM)
    out = _sc_gather()(inp2, tables_flat)
    return out.reshape(BATCH, NUM_FIELDS * EMB_DIM)
